# baseline (speedup 1.0000x reference)
import jax
import jax.numpy as jnp
from jax import lax
from jax.experimental import pallas as pl
from jax.experimental.pallas import tpu as pltpu
from jax.experimental.pallas import tpu_sc as plsc

VOCAB = 1000000
EMBED = 32
HALF = EMBED // 2
B = 4
S = 4096

NUM_CORES = 2
NUM_SUBCORES = 16
NW = NUM_CORES * NUM_SUBCORES
S_CHUNK = S // NW
GROUP = 16
NGROUP = (S_CHUNK // GROUP) * B
LANES = 128


def _sc_body(x_hbm, table_hbm, cs_hbm, out_hbm,
             idx_v, blk_v, out_v, cs_v, sem):
    cid = lax.axis_index("c")
    sid = lax.axis_index("s")
    wid = sid * NUM_CORES + cid
    s0 = wid * S_CHUNK

    pltpu.sync_copy(x_hbm, idx_v)
    pltpu.sync_copy(cs_hbm.at[pl.ds(s0, S_CHUNK)], cs_v)

    c_lo = lax.iota(jnp.int32, 16)
    c_hi = c_lo + HALF

    def group_body(g, carry):
        b = g // (S_CHUNK // GROUP)
        jc = g % (S_CHUNK // GROUP)
        toks = idx_v[pl.ds((b * (S // LANES) + wid) * LANES + jc * GROUP, GROUP)]
        bases = lax.bitwise_and(toks, jnp.full((GROUP,), ~127, jnp.int32))
        lanes = lax.bitwise_and(toks, jnp.full((GROUP,), 127, jnp.int32))
        copies = []
        for t in range(GROUP):
            r0 = pl.multiple_of(bases[t], 128)
            copies.append(
                pltpu.async_copy(
                    table_hbm.at[:, pl.ds(r0, LANES)], blk_v.at[t], sem
                )
            )
        for cp in copies:
            cp.wait()
        for t in range(GROUP):
            k = jc * GROUP + t
            uv = jnp.full((16,), lanes[t], dtype=jnp.int32)
            x1 = plsc.load_gather(blk_v.at[t], [c_lo, uv])
            x2 = plsc.load_gather(blk_v.at[t], [c_hi, uv])
            cv = cs_v[k, pl.ds(0, HALF)]
            sv = cs_v[k, pl.ds(HALF, HALF)]
            kv = jnp.full((16,), k, dtype=jnp.int32)
            plsc.store_scatter(out_v.at[b], [c_lo, kv], x1 * cv - x2 * sv)
            plsc.store_scatter(out_v.at[b], [c_hi, kv], x1 * sv + x2 * cv)
        return carry

    lax.fori_loop(0, NGROUP, group_body, 0)

    for b in range(B):
        pltpu.sync_copy(
            out_v.at[b], out_hbm.at[b, :, pl.ds(s0, S_CHUNK)]
        )


@jax.jit
def kernel(x, token_embedding):
    xf = x.reshape(B * S)
    table_t = token_embedding.T

    theta = 1.0 / (10000.0 ** (jnp.arange(HALF, dtype=jnp.float32) / HALF))
    ang = jnp.arange(S, dtype=jnp.float32)[:, None] * theta[None, :]
    cs_tab = jnp.concatenate(
        [jnp.cos(ang), jnp.sin(ang), jnp.zeros((S, LANES - EMBED), jnp.float32)],
        axis=1,
    )

    mesh = plsc.VectorSubcoreMesh(
        core_axis_name="c", subcore_axis_name="s",
        num_cores=NUM_CORES, num_subcores=NUM_SUBCORES,
    )
    run = pl.kernel(
        _sc_body,
        out_type=jax.ShapeDtypeStruct((B, EMBED, S), jnp.float32),
        mesh=mesh,
        scratch_types=[
            pltpu.VMEM((B * S,), jnp.int32),
            pltpu.VMEM((GROUP, EMBED, LANES), jnp.float32),
            pltpu.VMEM((B, EMBED, S_CHUNK), jnp.float32),
            pltpu.VMEM((S_CHUNK, LANES), jnp.float32),
            pltpu.SemaphoreType.DMA,
        ],
        compiler_params=pltpu.CompilerParams(needs_layout_passes=False),
    )
    packed = run(xf, table_t, cs_tab)
    return jnp.transpose(packed, (2, 0, 1))


# speedup vs baseline: 1.1647x; 1.1647x over previous
import jax
import jax.numpy as jnp
from jax import lax
from jax.experimental import pallas as pl
from jax.experimental.pallas import tpu as pltpu
from jax.experimental.pallas import tpu_sc as plsc

VOCAB = 1000000
EMBED = 32
HALF = EMBED // 2
B = 4
S = 4096

NUM_CORES = 2
NUM_SUBCORES = 16
NW = NUM_CORES * NUM_SUBCORES
S_CHUNK = S // NW
GROUP = 16
NGROUP = (S_CHUNK // GROUP) * B
LANES = 128


def _sc_body(x_hbm, table_hbm, cs_hbm, out_hbm,
             idx_v, blk_v, out_v, cs_v, sem):
    cid = lax.axis_index("c")
    sid = lax.axis_index("s")
    wid = sid * NUM_CORES + cid
    s0 = wid * S_CHUNK

    pltpu.sync_copy(x_hbm, idx_v)
    pltpu.sync_copy(cs_hbm.at[:, pl.ds(s0, S_CHUNK)], cs_v)

    c_lo = lax.iota(jnp.int32, 16)
    c_hi = c_lo + HALF

    def group_body(g, carry):
        b = g // (S_CHUNK // GROUP)
        jc = g % (S_CHUNK // GROUP)
        toks = idx_v[pl.ds((b * (S // LANES) + wid) * LANES + jc * GROUP, GROUP)]
        bases = lax.bitwise_and(toks, jnp.full((GROUP,), ~127, jnp.int32))
        lanes = lax.bitwise_and(toks, jnp.full((GROUP,), 127, jnp.int32))
        copies = []
        for t in range(GROUP):
            r0 = pl.multiple_of(bases[t], 128)
            copies.append(
                pltpu.async_copy(
                    table_hbm.at[:, pl.ds(r0, LANES)], blk_v.at[t], sem
                )
            )
        for cp in copies:
            cp.wait()
        for c in range(HALF):
            cv16 = jnp.full((16,), c, dtype=jnp.int32)
            x1 = plsc.load_gather(blk_v, [c_lo, cv16, lanes])
            x2 = plsc.load_gather(blk_v, [c_lo, cv16 + HALF, lanes])
            cv = cs_v[c, pl.ds(jc * GROUP, GROUP)]
            sv = cs_v[c + HALF, pl.ds(jc * GROUP, GROUP)]
            out_v[b, c, pl.ds(jc * GROUP, GROUP)] = x1 * cv - x2 * sv
            out_v[b, c + HALF, pl.ds(jc * GROUP, GROUP)] = x1 * sv + x2 * cv
        return carry

    lax.fori_loop(0, NGROUP, group_body, 0)

    for b in range(B):
        pltpu.sync_copy(
            out_v.at[b], out_hbm.at[b, :, pl.ds(s0, S_CHUNK)]
        )


@jax.jit
def kernel(x, token_embedding):
    xf = x.reshape(B * S)
    table_t = token_embedding.T

    theta = 1.0 / (10000.0 ** (jnp.arange(HALF, dtype=jnp.float32) / HALF))
    ang = jnp.arange(S, dtype=jnp.float32)[:, None] * theta[None, :]
    cs_tab = jnp.concatenate([jnp.cos(ang).T, jnp.sin(ang).T], axis=0)

    mesh = plsc.VectorSubcoreMesh(
        core_axis_name="c", subcore_axis_name="s",
        num_cores=NUM_CORES, num_subcores=NUM_SUBCORES,
    )
    run = pl.kernel(
        _sc_body,
        out_type=jax.ShapeDtypeStruct((B, EMBED, S), jnp.float32),
        mesh=mesh,
        scratch_types=[
            pltpu.VMEM((B * S,), jnp.int32),
            pltpu.VMEM((GROUP, EMBED, LANES), jnp.float32),
            pltpu.VMEM((B, EMBED, S_CHUNK), jnp.float32),
            pltpu.VMEM((EMBED, S_CHUNK), jnp.float32),
            pltpu.SemaphoreType.DMA,
        ],
        compiler_params=pltpu.CompilerParams(needs_layout_passes=False),
    )
    packed = run(xf, table_t, cs_tab)
    return jnp.transpose(packed, (2, 0, 1))
